# baseline (speedup 1.0000x reference)
import functools

import jax
import jax.numpy as jnp
from jax import lax
from jax.experimental import pallas as pl
from jax.experimental.pallas import tpu as pltpu
from jax.experimental.pallas import tpu_sc as plsc

_B = 32
_C = 768
_G = 4
_H = 24
_W = 24
_P = _B * _H * _W
_S = 10240
_NC = 2
_NS = 16
_NW = _NC * _NS
_RPW = _S // _NW
_CHUNK = 16
_NCHUNKS = _RPW // _CHUNK
_NBUF = 4
_L = 16
_NCB = _C // _L
_RB = 512
_TCB = (_P - _S) // _RB


@functools.partial(
    pl.kernel,
    mesh=plsc.VectorSubcoreMesh(core_axis_name="c", subcore_axis_name="s"),
    out_type=jax.ShapeDtypeStruct((_P, _C), jnp.float32),
    scratch_types=(
        [pltpu.VMEM((_CHUNK, _C), jnp.float32)] * (2 * _NBUF)
        + [pltpu.SemaphoreType.DMA] * (2 * _NBUF)
    ),
    compiler_params=pltpu.CompilerParams(
        use_tc_tiling_on_sc=False, needs_layout_passes=False),
)
def _channel_shuffle_sc(x_hbm, out_hbm, *refs):
    ins = refs[:_NBUF]
    outs = refs[_NBUF:2 * _NBUF]
    gsems = refs[2 * _NBUF:3 * _NBUF]
    ssems = refs[3 * _NBUF:4 * _NBUF]
    wid = lax.axis_index("s") * _NC + lax.axis_index("c")
    base = wid * _RPW

    iota4 = lax.iota(jnp.int32, _L) * 4
    idxs = [iota4 + g for g in range(_G)]

    _D = 8

    def permute_chunk(src, dst):
        blocks = [(g, t) for g in range(_G) for t in range(_NCB // _G)]

        def row_body(r, carry):
            row = src.at[r]
            orow = dst.at[r]
            vs = [None] * _NCB
            for i, (g, t) in enumerate(blocks):
                vs[i] = plsc.load_gather(
                    row.at[pl.ds(64 * t, 64)], [idxs[g]])
                if i >= _D:
                    j = i - _D
                    orow[pl.ds(j * _L, _L)] = vs[j]
            for j in range(_NCB - _D, _NCB):
                orow[pl.ds(j * _L, _L)] = vs[j]
            return carry
        lax.fori_loop(0, _CHUNK, row_body, 0)

    for b in range(_NBUF):
        pltpu.async_copy(
            x_hbm.at[pl.ds(base + b * _CHUNK, _CHUNK)], ins[b], gsems[b])

    def chunk_group(i, carry):
        for b in range(_NBUF):
            k = i * _NBUF + b
            row0 = base + k * _CHUNK
            pltpu.make_async_copy(
                x_hbm.at[pl.ds(row0, _CHUNK)], ins[b], gsems[b]).wait()

            @pl.when(k >= _NBUF)
            def _():
                pltpu.make_async_copy(
                    outs[b],
                    out_hbm.at[pl.ds(row0 - _NBUF * _CHUNK, _CHUNK)],
                    ssems[b]).wait()

            permute_chunk(ins[b], outs[b])
            pltpu.async_copy(
                outs[b], out_hbm.at[pl.ds(row0, _CHUNK)], ssems[b])

            @pl.when(k + _NBUF < _NCHUNKS)
            def _():
                pltpu.async_copy(
                    x_hbm.at[pl.ds(row0 + _NBUF * _CHUNK, _CHUNK)], ins[b],
                    gsems[b])
        return carry

    lax.fori_loop(0, _NCHUNKS // _NBUF, chunk_group, 0)
    for b in range(_NBUF):
        last = base + (_NCHUNKS - _NBUF + b) * _CHUNK
        pltpu.make_async_copy(
            outs[b], out_hbm.at[pl.ds(last, _CHUNK)], ssems[b]).wait()


def _tc_body(x_ref, o_ref):
    x = x_ref[...]
    idx = jax.lax.broadcasted_iota(jnp.int32, (_RB, 32), 1) * _G
    for k in range(_C // 128):
        win = x[:, 128 * k:128 * (k + 1)]
        for g in range(_G):
            piece = jnp.take_along_axis(win, idx + g, axis=1)
            c0 = (_C // _G) * g + 32 * k
            o_ref[:, c0:c0 + 32] = piece


def _channel_shuffle_tc(x2d):
    return pl.pallas_call(
        _tc_body,
        grid=(_TCB,),
        in_specs=[pl.BlockSpec((_RB, _C), lambda i: (i + _S // _RB, 0))],
        out_specs=pl.BlockSpec((_RB, _C), lambda i: (i, 0)),
        out_shape=jax.ShapeDtypeStruct((_P - _S, _C), jnp.float32),
    )(x2d)


def kernel(input):
    x2d = input.transpose(0, 2, 3, 1).reshape(_P, _C)
    out_sc = _channel_shuffle_sc(x2d)
    out_tc = _channel_shuffle_tc(x2d)
    out = lax.dynamic_update_slice(out_sc, out_tc, (_S, 0))
    return out.reshape(_B, _H, _W, _C).transpose(0, 3, 1, 2)


# speedup vs baseline: 1.1901x; 1.1901x over previous
import functools

import jax
import jax.numpy as jnp
from jax import lax
from jax.experimental import pallas as pl
from jax.experimental.pallas import tpu as pltpu
from jax.experimental.pallas import tpu_sc as plsc

_B = 32
_C = 768
_G = 4
_H = 24
_W = 24
_P = _B * _H * _W
_NC = 2
_NS = 16
_NW = _NC * _NS
_RPW = _P // _NW
_CHUNK = 16
_NCHUNKS = _RPW // _CHUNK
_NBUF = 4
_L = 16
_NCB = _C // _L


@functools.partial(
    pl.kernel,
    mesh=plsc.VectorSubcoreMesh(core_axis_name="c", subcore_axis_name="s"),
    out_type=jax.ShapeDtypeStruct((_P, _C), jnp.float32),
    scratch_types=(
        [pltpu.VMEM((_CHUNK, _C), jnp.float32)] * (2 * _NBUF)
        + [pltpu.SemaphoreType.DMA] * (2 * _NBUF)
    ),
    compiler_params=pltpu.CompilerParams(
        use_tc_tiling_on_sc=False, needs_layout_passes=False),
)
def _channel_shuffle_sc(x_hbm, out_hbm, *refs):
    ins = refs[:_NBUF]
    outs = refs[_NBUF:2 * _NBUF]
    gsems = refs[2 * _NBUF:3 * _NBUF]
    ssems = refs[3 * _NBUF:4 * _NBUF]
    wid = lax.axis_index("s") * _NC + lax.axis_index("c")
    base = wid * _RPW

    iota4 = lax.iota(jnp.int32, _L) * 4
    idxs = [iota4 + g for g in range(_G)]

    _D = 8

    def permute_chunk(src, dst):
        blocks = [(g, t) for g in range(_G) for t in range(_NCB // _G)]

        def row_body(r, carry):
            row = src.at[r]
            orow = dst.at[r]
            vs = [None] * _NCB
            for i, (g, t) in enumerate(blocks):
                vs[i] = plsc.load_gather(
                    row.at[pl.ds(64 * t, 64)], [idxs[g]])
                if i >= _D:
                    j = i - _D
                    orow[pl.ds(j * _L, _L)] = vs[j]
            for j in range(_NCB - _D, _NCB):
                orow[pl.ds(j * _L, _L)] = vs[j]
            return carry
        lax.fori_loop(0, _CHUNK, row_body, 0)

    for b in range(_NBUF):
        pltpu.async_copy(
            x_hbm.at[pl.ds(base + b * _CHUNK, _CHUNK)], ins[b], gsems[b])

    def chunk_group(i, carry):
        for b in range(_NBUF):
            k = i * _NBUF + b
            row0 = base + k * _CHUNK
            pltpu.make_async_copy(
                x_hbm.at[pl.ds(row0, _CHUNK)], ins[b], gsems[b]).wait()

            @pl.when(k >= _NBUF)
            def _():
                pltpu.make_async_copy(
                    outs[b],
                    out_hbm.at[pl.ds(row0 - _NBUF * _CHUNK, _CHUNK)],
                    ssems[b]).wait()

            permute_chunk(ins[b], outs[b])
            pltpu.async_copy(
                outs[b], out_hbm.at[pl.ds(row0, _CHUNK)], ssems[b])

            @pl.when(k + _NBUF < _NCHUNKS)
            def _():
                pltpu.async_copy(
                    x_hbm.at[pl.ds(row0 + _NBUF * _CHUNK, _CHUNK)], ins[b],
                    gsems[b])
        return carry

    lax.fori_loop(0, _NCHUNKS // _NBUF, chunk_group, 0)
    for b in range(_NBUF):
        last = base + (_NCHUNKS - _NBUF + b) * _CHUNK
        pltpu.make_async_copy(
            outs[b], out_hbm.at[pl.ds(last, _CHUNK)], ssems[b]).wait()


def kernel(input):
    x2d = input.transpose(0, 2, 3, 1).reshape(_P, _C)
    out = _channel_shuffle_sc(x2d)
    return out.reshape(_B, _H, _W, _C).transpose(0, 3, 1, 2)


# speedup vs baseline: 1.1936x; 1.0030x over previous
import functools

import jax
import jax.numpy as jnp
from jax import lax
from jax.experimental import pallas as pl
from jax.experimental.pallas import tpu as pltpu
from jax.experimental.pallas import tpu_sc as plsc

_B = 32
_C = 768
_G = 4
_H = 24
_W = 24
_P = _B * _H * _W
_NC = 2
_NS = 16
_NW = _NC * _NS
_RPW = _P // _NW
_CHUNK = 8
_NCHUNKS = _RPW // _CHUNK
_NBUF = 8
_L = 16
_NCB = _C // _L


@functools.partial(
    pl.kernel,
    mesh=plsc.VectorSubcoreMesh(core_axis_name="c", subcore_axis_name="s"),
    out_type=jax.ShapeDtypeStruct((_P, _C), jnp.float32),
    scratch_types=(
        [pltpu.VMEM((_CHUNK, _C), jnp.float32)] * (2 * _NBUF)
        + [pltpu.SemaphoreType.DMA] * (2 * _NBUF)
    ),
    compiler_params=pltpu.CompilerParams(
        use_tc_tiling_on_sc=False, needs_layout_passes=False),
)
def _channel_shuffle_sc(x_hbm, out_hbm, *refs):
    ins = refs[:_NBUF]
    outs = refs[_NBUF:2 * _NBUF]
    gsems = refs[2 * _NBUF:3 * _NBUF]
    ssems = refs[3 * _NBUF:4 * _NBUF]
    wid = lax.axis_index("s") * _NC + lax.axis_index("c")
    base = wid * _RPW

    iota4 = lax.iota(jnp.int32, _L) * 4
    idxs = [iota4 + g for g in range(_G)]

    _D = 8

    def permute_chunk(src, dst):
        blocks = [(g, t) for g in range(_G) for t in range(_NCB // _G)]

        def row_body(r, carry):
            row = src.at[r]
            orow = dst.at[r]
            vs = [None] * _NCB
            for i, (g, t) in enumerate(blocks):
                vs[i] = plsc.load_gather(
                    row.at[pl.ds(64 * t, 64)], [idxs[g]])
                if i >= _D:
                    j = i - _D
                    orow[pl.ds(j * _L, _L)] = vs[j]
            for j in range(_NCB - _D, _NCB):
                orow[pl.ds(j * _L, _L)] = vs[j]
            return carry
        lax.fori_loop(0, _CHUNK, row_body, 0)

    for b in range(_NBUF):
        pltpu.async_copy(
            x_hbm.at[pl.ds(base + b * _CHUNK, _CHUNK)], ins[b], gsems[b])

    def chunk_group(i, carry):
        for b in range(_NBUF):
            k = i * _NBUF + b
            row0 = base + k * _CHUNK
            pltpu.make_async_copy(
                x_hbm.at[pl.ds(row0, _CHUNK)], ins[b], gsems[b]).wait()

            @pl.when(k >= _NBUF)
            def _():
                pltpu.make_async_copy(
                    outs[b],
                    out_hbm.at[pl.ds(row0 - _NBUF * _CHUNK, _CHUNK)],
                    ssems[b]).wait()

            permute_chunk(ins[b], outs[b])
            pltpu.async_copy(
                outs[b], out_hbm.at[pl.ds(row0, _CHUNK)], ssems[b])

            @pl.when(k + _NBUF < _NCHUNKS)
            def _():
                pltpu.async_copy(
                    x_hbm.at[pl.ds(row0 + _NBUF * _CHUNK, _CHUNK)], ins[b],
                    gsems[b])
        return carry

    lax.fori_loop(0, _NCHUNKS // _NBUF, chunk_group, 0)
    for b in range(_NBUF):
        last = base + (_NCHUNKS - _NBUF + b) * _CHUNK
        pltpu.make_async_copy(
            outs[b], out_hbm.at[pl.ds(last, _CHUNK)], ssems[b]).wait()


def kernel(input):
    x2d = input.transpose(0, 2, 3, 1).reshape(_P, _C)
    out = _channel_shuffle_sc(x2d)
    return out.reshape(_B, _H, _W, _C).transpose(0, 3, 1, 2)
